# baseline (speedup 1.0000x reference)
import jax
import jax.numpy as jnp
from jax.experimental import pallas as pl
from jax.experimental.pallas import tpu as pltpu
from jax.experimental.pallas import tpu_sc as plsc

DIM = 128
H = DIM // 2
N = 10000
K = 8
BN = 400
_CHUNKS = 5

F32 = jnp.float32
BF16 = jnp.bfloat16


def _mm(x, w):
    return jax.lax.dot_general(x, w, (((1,), (1,)), ((), ())),
                               preferred_element_type=F32)


def _pack_lstm(wih, whh, bih, bhh):
    bias = bih + bhh
    zero = jnp.zeros((H, DIM), wih.dtype)
    row_blocks = []
    for gi, g in enumerate((0, 1, 3, 2)):
        scale = 1.0 if g == 2 else 0.5
        for d in range(2):
            blk = slice(g * H, (g + 1) * H)
            wx = wih[d, blk, :] * scale
            cols_x = [wx, zero] if d == 0 else [zero, wx]
            wh = jnp.pad(whh[d, blk, :] * scale,
                         ((0, 0), (d * H, (1 - d) * H)))
            wb = jnp.broadcast_to(bias[d, blk, None] * (scale / DIM),
                                  (H, DIM))
            row_blocks.append(jnp.concatenate(cols_x + [wh, wb], axis=1))
    return jnp.concatenate(row_blocks, axis=0).astype(BF16)


def _lstm_packed(xs, w):
    kl = len(xs)
    B = xs[0].shape[0]
    ones = jnp.ones((B, DIM), BF16)
    zeros16 = jnp.zeros((B, DIM), BF16)
    c = None
    h16 = zeros16
    hsum = None
    for s in range(kl):
        xcat = jnp.concatenate([xs[s], xs[kl - 1 - s], h16, ones], axis=1)
        t = jnp.tanh(_mm(xcat, w))
        si = 0.5 * t[:, :DIM] + 0.5
        tg = t[:, 3 * DIM:]
        if c is None:
            c = si * tg
        else:
            sf = 0.5 * t[:, DIM:2 * DIM] + 0.5
            c = sf * c + si * tg
        so = 0.5 * t[:, 2 * DIM:3 * DIM] + 0.5
        h = so * jnp.tanh(c)
        hsum = h if hsum is None else hsum + h
        if s < kl - 1:
            h16 = h.astype(BF16)
    return hsum * (1.0 / kl)


BN1 = 1000


def _content_body(a0_ref, a1_ref, b0_ref, b1_ref, w_ref, out_ref):
    x0 = jnp.concatenate([a0_ref[...], b0_ref[...]], axis=0).astype(BF16)
    x1 = jnp.concatenate([a1_ref[...], b1_ref[...]], axis=0).astype(BF16)
    enc = _lstm_packed([x0, x1], w_ref[...])
    out_ref[0] = enc[:BN1]
    out_ref[1] = enc[BN1:]


def _content_encode(a0, a1, b0, b1, w, *, interpret=False):
    grid = (N // BN1,)
    row = pl.BlockSpec((BN1, DIM), lambda i: (i, 0))

    def full(x):
        return pl.BlockSpec(x.shape, lambda i, _nd=x.ndim: (0,) * _nd)

    return pl.pallas_call(
        _content_body,
        grid=grid,
        in_specs=[row, row, row, row, full(w)],
        out_specs=pl.BlockSpec((2, BN1, DIM), lambda i: (0, i, 0)),
        out_shape=jax.ShapeDtypeStruct((2, N, DIM), F32),
        compiler_params=pltpu.CompilerParams(
            dimension_semantics=("parallel",)),
        interpret=interpret,
    )(a0, a1, b0, b1, w)


_GATHER_WINDOW = 256


def _sc_gather(table, idx):
    n_idx = idx.shape[0]
    idx = idx.reshape(1, n_idx)
    mesh = plsc.VectorSubcoreMesh(core_axis_name="core",
                                  subcore_axis_name="subcore")

    @pl.kernel(out_type=jax.ShapeDtypeStruct((n_idx, DIM), table.dtype),
               mesh=mesh)
    def gather_kernel(tab_hbm, i_hbm, o_hbm):
        def body(i_vmem, o_vmem):
            pltpu.sync_copy(tab_hbm.at[i_vmem.at[0]], o_vmem)

        pltpu.emit_pipeline(
            body,
            grid=(n_idx // _GATHER_WINDOW,),
            in_specs=[pl.BlockSpec((1, _GATHER_WINDOW), lambda i: (0, i))],
            out_specs=[pl.BlockSpec((_GATHER_WINDOW, DIM),
                                    lambda i: (i, 0))],
            core_axis_name=("core", "subcore"),
            dimension_semantics=(pltpu.PARALLEL,),
        )(i_hbm, o_hbm)

    return gather_kernel(table, idx)


def _agg_body(m_ref, content_ref, aw_ref, bw_ref, attwa_ref, attba_ref,
              attwb_ref, attbb_ref, out_ref):
    aggs = [None] * 4
    for pair, w_ref in (((0, 2), aw_ref), ((1, 3), bw_ref)):
        m0 = m_ref[pair[0]]
        m1 = m_ref[pair[1]]
        xs = [jnp.concatenate([m0[:, t * DIM:(t + 1) * DIM],
                               m1[:, t * DIM:(t + 1) * DIM]],
                              axis=0).astype(BF16) for t in range(K)]
        enc = _lstm_packed(xs, w_ref[...])
        aggs[pair[0]] = enc[:BN]
        aggs[pair[1]] = enc[BN:]

    def attend(d, e0, e1, attw, attb):
        w1 = attw[0, :DIM].reshape(1, DIM)
        w2 = attw[0, DIM:].reshape(1, DIM)
        b = attb[0, 0]
        dw = jnp.sum(d * w1, axis=1, keepdims=True)

        def score(e):
            s = dw + jnp.sum(e * w2, axis=1, keepdims=True) + b
            return jnp.where(s >= 0, s, 0.01 * s)

        s0, s1, s2 = score(e0), score(e1), score(d)
        m = jnp.maximum(jnp.maximum(s0, s1), s2)
        z0 = jnp.exp(s0 - m)
        z1 = jnp.exp(s1 - m)
        z2 = jnp.exp(s2 - m)
        inv = 1.0 / (z0 + z1 + z2)
        return (z0 * e0 + z1 * e1 + z2 * d) * inv

    out_ref[0] = attend(content_ref[0], aggs[0], aggs[1],
                        attwa_ref[...], attba_ref[...])
    out_ref[1] = attend(content_ref[1], aggs[2], aggs[3],
                        attwb_ref[...], attbb_ref[...])


def _agg_attend(msgs, content, aw, bw,
                attwa, attba, attwb, attbb, node0=0, nnodes=N, *,
                interpret=False):
    grid = (nnodes // BN,)
    blk0 = node0 // BN

    def full(x):
        return pl.BlockSpec(x.shape, lambda i, _nd=x.ndim: (0,) * _nd)

    return pl.pallas_call(
        _agg_body,
        grid=grid,
        in_specs=[
            pl.BlockSpec((4, BN, K * DIM), lambda i: (0, i, 0)),
            pl.BlockSpec((2, BN, DIM), lambda i: (0, blk0 + i, 0)),
            full(aw), full(bw),
            full(attwa), full(attba), full(attwb), full(attbb),
        ],
        out_specs=pl.BlockSpec((2, BN, DIM), lambda i: (0, i, 0)),
        out_shape=jax.ShapeDtypeStruct((2, nnodes, DIM), F32),
        compiler_params=pltpu.CompilerParams(
            dimension_semantics=("parallel",)),
        interpret=interpret,
    )(msgs, content, aw, bw, attwa, attba, attwb, attbb)


def kernel(h_A_c0, h_A_c1, h_B_c0, h_B_c1, neigh_AA, neigh_BA, neigh_AB,
           neigh_BB, c_Wih, c_Whh, c_bih, c_bhh, nA_Wih, nA_Whh, nA_bih,
           nA_bhh, nB_Wih, nB_Whh, nB_bih, nB_bhh, attW_A, attb_A, attW_B,
           attb_B):
    cw = _pack_lstm(c_Wih, c_Whh, c_bih, c_bhh)
    aw = _pack_lstm(nA_Wih, nA_Whh, nA_bih, nA_bhh)
    bw = _pack_lstm(nB_Wih, nB_Whh, nB_bih, nB_bhh)

    content = _content_encode(h_A_c0, h_A_c1, h_B_c0, h_B_c1, cw)

    idx = jnp.stack([
        neigh_AA,
        neigh_BA + N,
        neigh_AB,
        neigh_BB + N,
    ])
    table = content.reshape(2 * N, DIM)

    nc = N // _CHUNKS
    outs = []
    for g in range(_CHUNKS):
        idx_g = idx[:, g * nc:(g + 1) * nc, :].reshape(-1)
        msgs_g = _sc_gather(table, idx_g).reshape(4, nc, K * DIM)
        outs.append(_agg_attend(msgs_g, content, aw, bw,
                                attW_A, attb_A.reshape(1, 1),
                                attW_B, attb_B.reshape(1, 1),
                                node0=g * nc, nnodes=nc))
    return jnp.concatenate(outs, axis=1)


# speedup vs baseline: 1.6023x; 1.6023x over previous
import jax
import jax.numpy as jnp
from jax.experimental import pallas as pl
from jax.experimental.pallas import tpu as pltpu
from jax.experimental.pallas import tpu_sc as plsc

DIM = 128
H = DIM // 2
N = 10000
K = 8
BN = 400
_CHUNKS = 5

F32 = jnp.float32
BF16 = jnp.bfloat16


def _mm(x, w):
    return jax.lax.dot_general(x, w, (((1,), (1,)), ((), ())),
                               preferred_element_type=F32)


def _pack_lstm(wih, whh, bih, bhh):
    bias = bih + bhh
    zero = jnp.zeros((H, DIM), wih.dtype)
    row_blocks = []
    for gi, g in enumerate((0, 1, 3, 2)):
        scale = 1.0 if g == 2 else 0.5
        for d in range(2):
            blk = slice(g * H, (g + 1) * H)
            wx = wih[d, blk, :] * scale
            cols_x = [wx, zero] if d == 0 else [zero, wx]
            wh = jnp.pad(whh[d, blk, :] * scale,
                         ((0, 0), (d * H, (1 - d) * H)))
            wb = jnp.broadcast_to(bias[d, blk, None] * (scale / DIM),
                                  (H, DIM))
            row_blocks.append(jnp.concatenate(cols_x + [wh, wb], axis=1))
    return jnp.concatenate(row_blocks, axis=0).astype(BF16)


def _lstm_packed(xs, w):
    kl = len(xs)
    B = xs[0].shape[0]
    ones = jnp.ones((B, DIM), BF16)
    zeros16 = jnp.zeros((B, DIM), BF16)
    c = None
    h16 = zeros16
    hsum = None
    for s in range(kl):
        xcat = jnp.concatenate([xs[s], xs[kl - 1 - s], h16, ones], axis=1)
        t = jnp.tanh(_mm(xcat, w))
        si = 0.5 * t[:, :DIM] + 0.5
        tg = t[:, 3 * DIM:]
        if c is None:
            c = si * tg
        else:
            sf = 0.5 * t[:, DIM:2 * DIM] + 0.5
            c = sf * c + si * tg
        so = 0.5 * t[:, 2 * DIM:3 * DIM] + 0.5
        h = so * jnp.tanh(c)
        hsum = h if hsum is None else hsum + h
        if s < kl - 1:
            h16 = h.astype(BF16)
    return hsum * (1.0 / kl)


BN1 = 1000


def _content_body(a0_ref, a1_ref, b0_ref, b1_ref, w_ref, out_ref):
    x0 = jnp.concatenate([a0_ref[...], b0_ref[...]], axis=0).astype(BF16)
    x1 = jnp.concatenate([a1_ref[...], b1_ref[...]], axis=0).astype(BF16)
    enc = _lstm_packed([x0, x1], w_ref[...])
    out_ref[0] = enc[:BN1]
    out_ref[1] = enc[BN1:]


def _content_encode(a0, a1, b0, b1, w, *, interpret=False):
    grid = (N // BN1,)
    row = pl.BlockSpec((BN1, DIM), lambda i: (i, 0))

    def full(x):
        return pl.BlockSpec(x.shape, lambda i, _nd=x.ndim: (0,) * _nd)

    return pl.pallas_call(
        _content_body,
        grid=grid,
        in_specs=[row, row, row, row, full(w)],
        out_specs=pl.BlockSpec((2, BN1, DIM), lambda i: (0, i, 0)),
        out_shape=jax.ShapeDtypeStruct((2, N, DIM), F32),
        compiler_params=pltpu.CompilerParams(
            dimension_semantics=("parallel",)),
        interpret=interpret,
    )(a0, a1, b0, b1, w)


_GATHER_WINDOW = 256


def _sc_gather(table, idx):
    n_idx = idx.shape[0]
    idx = idx.reshape(1, n_idx)
    mesh = plsc.VectorSubcoreMesh(core_axis_name="core",
                                  subcore_axis_name="subcore")

    @pl.kernel(out_type=jax.ShapeDtypeStruct((n_idx, DIM), table.dtype),
               mesh=mesh)
    def gather_kernel(tab_hbm, i_hbm, o_hbm):
        def body(i_vmem, o_vmem):
            pltpu.sync_copy(tab_hbm.at[i_vmem.at[0]], o_vmem)

        pltpu.emit_pipeline(
            body,
            grid=(n_idx // _GATHER_WINDOW,),
            in_specs=[pl.BlockSpec((1, _GATHER_WINDOW), lambda i: (0, i))],
            out_specs=[pl.BlockSpec((_GATHER_WINDOW, DIM),
                                    lambda i: (i, 0))],
            core_axis_name=("core", "subcore"),
            dimension_semantics=(pltpu.PARALLEL,),
        )(i_hbm, o_hbm)

    return gather_kernel(table, idx)


def _agg_body(m_ref, content_ref, aw_ref, bw_ref, attwa_ref, attba_ref,
              attwb_ref, attbb_ref, out_ref):
    aggs = [None] * 4
    for pair, w_ref in (((0, 2), aw_ref), ((1, 3), bw_ref)):
        xs = [jnp.concatenate([m_ref[pair[0], t], m_ref[pair[1], t]],
                              axis=0).astype(BF16) for t in range(K)]
        enc = _lstm_packed(xs, w_ref[...])
        aggs[pair[0]] = enc[:BN]
        aggs[pair[1]] = enc[BN:]

    def attend(d, e0, e1, attw, attb):
        w1 = attw[0, :DIM].reshape(1, DIM)
        w2 = attw[0, DIM:].reshape(1, DIM)
        b = attb[0, 0]
        dw = jnp.sum(d * w1, axis=1, keepdims=True)

        def score(e):
            s = dw + jnp.sum(e * w2, axis=1, keepdims=True) + b
            return jnp.where(s >= 0, s, 0.01 * s)

        s0, s1, s2 = score(e0), score(e1), score(d)
        m = jnp.maximum(jnp.maximum(s0, s1), s2)
        z0 = jnp.exp(s0 - m)
        z1 = jnp.exp(s1 - m)
        z2 = jnp.exp(s2 - m)
        inv = 1.0 / (z0 + z1 + z2)
        return (z0 * e0 + z1 * e1 + z2 * d) * inv

    out_ref[0] = attend(content_ref[0], aggs[0], aggs[1],
                        attwa_ref[...], attba_ref[...])
    out_ref[1] = attend(content_ref[1], aggs[2], aggs[3],
                        attwb_ref[...], attbb_ref[...])


def _agg_attend(msgs, content, aw, bw,
                attwa, attba, attwb, attbb, node0=0, nnodes=N, *,
                interpret=False):
    grid = (nnodes // BN,)
    blk0 = node0 // BN

    def full(x):
        return pl.BlockSpec(x.shape, lambda i, _nd=x.ndim: (0,) * _nd)

    return pl.pallas_call(
        _agg_body,
        grid=grid,
        in_specs=[
            pl.BlockSpec((4, K, BN, DIM), lambda i: (0, 0, i, 0)),
            pl.BlockSpec((2, BN, DIM), lambda i: (0, blk0 + i, 0)),
            full(aw), full(bw),
            full(attwa), full(attba), full(attwb), full(attbb),
        ],
        out_specs=pl.BlockSpec((2, BN, DIM), lambda i: (0, i, 0)),
        out_shape=jax.ShapeDtypeStruct((2, nnodes, DIM), F32),
        compiler_params=pltpu.CompilerParams(
            dimension_semantics=("parallel",)),
        interpret=interpret,
    )(msgs, content, aw, bw, attwa, attba, attwb, attbb)


def kernel(h_A_c0, h_A_c1, h_B_c0, h_B_c1, neigh_AA, neigh_BA, neigh_AB,
           neigh_BB, c_Wih, c_Whh, c_bih, c_bhh, nA_Wih, nA_Whh, nA_bih,
           nA_bhh, nB_Wih, nB_Whh, nB_bih, nB_bhh, attW_A, attb_A, attW_B,
           attb_B):
    cw = _pack_lstm(c_Wih, c_Whh, c_bih, c_bhh)
    aw = _pack_lstm(nA_Wih, nA_Whh, nA_bih, nA_bhh)
    bw = _pack_lstm(nB_Wih, nB_Whh, nB_bih, nB_bhh)

    content = _content_encode(h_A_c0, h_A_c1, h_B_c0, h_B_c1, cw)

    idx = jnp.stack([
        neigh_AA.T,
        neigh_BA.T + N,
        neigh_AB.T,
        neigh_BB.T + N,
    ])
    table = content.reshape(2 * N, DIM)

    nc = N // _CHUNKS
    outs = []
    for g in range(_CHUNKS):
        idx_g = idx[:, :, g * nc:(g + 1) * nc].reshape(-1)
        msgs_g = _sc_gather(table, idx_g).reshape(4, K, nc, DIM)
        outs.append(_agg_attend(msgs_g, content, aw, bw,
                                attW_A, attb_A.reshape(1, 1),
                                attW_B, attb_B.reshape(1, 1),
                                node0=g * nc, nnodes=nc))
    return jnp.concatenate(outs, axis=1)
